# Initial kernel scaffold; baseline (speedup 1.0000x reference)
import jax
import jax.numpy as jnp
from jax.experimental import pallas as pl


def kernel(x, edge_index, W1, b1, W_mu, b_mu, W_logstd, b_logstd):
    raise NotImplementedError("write your pallas kernel here")


# speedup vs baseline: 27.4311x; 27.4311x over previous
import functools

import jax
import jax.numpy as jnp
from jax import lax
from jax.experimental import pallas as pl
from jax.experimental.pallas import tpu as pltpu
from jax.experimental.pallas import tpu_sc as plsc

N = 10000
E = 320000
D = 128
NC = 2
NS = 16
NW = NC * NS
CHUNK = 128
NCH = 80
EPW = NCH * CHUNK
E2 = NW * EPW
PAD = E2 - E
NACC = 10112
NPADROW = NACC - N
RPS = NACC // NS
NPH = 2
CPP = NCH // NPH

_MESH = plsc.VectorSubcoreMesh(core_axis_name="c", subcore_axis_name="s")


def _sc_degree_hist(dst2, ones_blk, zeros_hist):
    @functools.partial(
        pl.kernel,
        out_type=jax.ShapeDtypeStruct((NC, NACC, 128), jnp.float32),
        mesh=_MESH,
        scratch_types=[
            pltpu.VMEM((NCH, CHUNK), jnp.int32),
            pltpu.VMEM((CHUNK, 128), jnp.float32),
            pltpu.VMEM_SHARED((NACC, 128), jnp.float32),
            pltpu.SemaphoreType.DMA,
            pltpu.SemaphoreType.DMA,
        ],
    )
    def k(dst_hbm, ones_hbm, zeros_hbm, out_hbm, dstv, ones_v, acc, sa, sb):
        cid = lax.axis_index("c")
        sid = lax.axis_index("s")
        wid = sid * NC + cid
        pltpu.sync_copy(dst_hbm.at[pl.ds(wid * NCH, NCH)], dstv)
        pltpu.sync_copy(ones_hbm, ones_v)
        pltpu.sync_copy(zeros_hbm, acc.at[pl.ds(sid * RPS, RPS)])
        plsc.subcore_barrier()

        pltpu.async_copy(ones_v, acc.at[dstv.at[0]], sa, add=True)

        @pl.loop(0, NCH, step=2)
        def _(j):
            pltpu.async_copy(ones_v, acc.at[dstv.at[j + 1]], sb, add=True)
            pltpu.make_async_copy(ones_v, acc.at[dstv.at[j]], sa).wait()

            @pl.when(j + 2 < NCH)
            def _():
                pltpu.async_copy(ones_v, acc.at[dstv.at[j + 2]], sa, add=True)

            pltpu.make_async_copy(ones_v, acc.at[dstv.at[j + 1]], sb).wait()

        plsc.subcore_barrier()
        pltpu.sync_copy(acc.at[pl.ds(sid * RPS, RPS)],
                        out_hbm.at[cid, pl.ds(sid * RPS, RPS)])

    return k(dst2, ones_blk, zeros_hist)


def _sc_propagate(u, src2, dst2, zeros_blk):
    @functools.partial(
        pl.kernel,
        out_type=jax.ShapeDtypeStruct((NC, NACC, D), jnp.float32),
        mesh=_MESH,
        scratch_types=[
            pltpu.VMEM((CPP, CHUNK), jnp.int32),
            pltpu.VMEM((CPP, CHUNK), jnp.int32),
            pltpu.VMEM((CHUNK, D), jnp.float32),
            pltpu.VMEM((CHUNK, D), jnp.float32),
            pltpu.VMEM_SHARED((NACC, D), jnp.float32),
            pltpu.SemaphoreType.DMA,
            pltpu.SemaphoreType.DMA,
        ],
    )
    def k(u_hbm, src_hbm, dst_hbm, z_hbm, out_hbm,
          srcv, dstv, ra, rb, acc, sa, sb):
        cid = lax.axis_index("c")
        sid = lax.axis_index("s")
        wid = sid * NC + cid
        for t in range(4):
            pltpu.sync_copy(z_hbm, acc.at[pl.ds(sid * RPS + t * CHUNK, CHUNK)])
        pltpu.sync_copy(z_hbm.at[pl.ds(0, RPS - 4 * CHUNK)],
                        acc.at[pl.ds(sid * RPS + 4 * CHUNK, RPS - 4 * CHUNK)])
        plsc.subcore_barrier()

        for ph in range(NPH):
            base = wid * NCH + ph * CPP
            pltpu.sync_copy(src_hbm.at[pl.ds(base, CPP)], srcv)
            pltpu.sync_copy(dst_hbm.at[pl.ds(base, CPP)], dstv)
            pltpu.async_copy(u_hbm.at[srcv.at[0]], ra, sa)

            @pl.loop(0, CPP, step=2)
            def _(j):
                pltpu.make_async_copy(u_hbm.at[srcv.at[j]], ra, sa).wait()
                pltpu.async_copy(u_hbm.at[srcv.at[j + 1]], rb, sb)
                pltpu.sync_copy(ra, acc.at[dstv.at[j]], add=True)
                pltpu.make_async_copy(u_hbm.at[srcv.at[j + 1]], rb, sb).wait()

                @pl.when(j + 2 < CPP)
                def _():
                    pltpu.async_copy(u_hbm.at[srcv.at[j + 2]], ra, sa)

                pltpu.sync_copy(rb, acc.at[dstv.at[j + 1]], add=True)

        plsc.subcore_barrier()
        pltpu.sync_copy(acc.at[pl.ds(sid * RPS, RPS)],
                        out_hbm.at[cid, pl.ds(sid * RPS, RPS)])

    return k(u, src2, dst2, zeros_blk)


def _dot(a, b):
    return lax.dot_general(a, b, (((1,), (0,)), ((), ())),
                           precision=lax.Precision.HIGHEST,
                           preferred_element_type=jnp.float32)


def _tc_matmul(x, w):
    def body(x_ref, w_ref, o_ref):
        o_ref[...] = _dot(x_ref[...], w_ref[...])

    return pl.pallas_call(
        body,
        out_shape=jax.ShapeDtypeStruct((x.shape[0], w.shape[1]), jnp.float32),
    )(x, w)


def _tc_scale1(hist, t1):
    def body(h_ref, t_ref, u_ref, d_ref):
        cnt = h_ref[0, :N, 0:1] + h_ref[1, :N, 0:1]
        d = lax.rsqrt(cnt + 1.0)
        d_ref[...] = d
        u_ref[...] = t_ref[...] * d

    return pl.pallas_call(
        body,
        out_shape=(
            jax.ShapeDtypeStruct((N, D), jnp.float32),
            jax.ShapeDtypeStruct((N, 1), jnp.float32),
        ),
    )(hist, t1)


def _tc_mid(p1, u1, dcol, w23, b1):
    def body(p_ref, u_ref, d_ref, w_ref, b_ref, o_ref):
        s = p_ref[0, :N] + p_ref[1, :N] + u_ref[...]
        h = jnp.maximum(s * d_ref[...] + b_ref[...], 0.0)
        o_ref[...] = _dot(h, w_ref[...]) * d_ref[...]

    return pl.pallas_call(
        body,
        out_shape=jax.ShapeDtypeStruct((N, D), jnp.float32),
    )(p1, u1, dcol, w23, b1)


def _tc_post(p2, u2, dcol, bmu, bls):
    def body(p_ref, u_ref, d_ref, bm_ref, bl_ref, mu_ref, ls_ref):
        o = (p_ref[0, :N] + p_ref[1, :N] + u_ref[...]) * d_ref[...]
        mu_ref[...] = o[:, : D // 2] + bm_ref[...]
        ls_ref[...] = o[:, D // 2:] + bl_ref[...]

    return pl.pallas_call(
        body,
        out_shape=(
            jax.ShapeDtypeStruct((N, D // 2), jnp.float32),
            jax.ShapeDtypeStruct((N, D // 2), jnp.float32),
        ),
    )(p2, u2, dcol, bmu, bls)


def kernel(x, edge_index, W1, b1, W_mu, b_mu, W_logstd, b_logstd):
    ei = edge_index.astype(jnp.int32)
    src, dst = ei[0], ei[1]
    r = jnp.arange(PAD, dtype=jnp.int32)
    src2 = jnp.concatenate([src, (r * 97) % N]).reshape(NW * NCH, CHUNK)
    dst2 = jnp.concatenate([dst, N + (r % NPADROW)]).reshape(NW * NCH, CHUNK)

    ones_blk = jnp.ones((CHUNK, 128), jnp.float32)
    zeros_hist = jnp.zeros((RPS, 128), jnp.float32)
    zeros_blk = jnp.zeros((CHUNK, D), jnp.float32)

    hist = _sc_degree_hist(dst2, ones_blk, zeros_hist)
    t1 = _tc_matmul(x, W1)
    u1, dcol = _tc_scale1(hist, t1)
    p1 = _sc_propagate(u1, src2, dst2, zeros_blk)
    w23 = jnp.concatenate([W_mu, W_logstd], axis=1)
    u2 = _tc_mid(p1, u1, dcol, w23, b1.reshape(1, D))
    p2 = _sc_propagate(u2, src2, dst2, zeros_blk)
    mu, logstd = _tc_post(p2, u2, dcol, b_mu.reshape(1, D // 2),
                          b_logstd.reshape(1, D // 2))
    return (mu, logstd)
